# baseline (speedup 1.0000x reference)
import functools

import jax
import jax.numpy as jnp
from jax import lax
from jax.experimental import pallas as pl
from jax.experimental.pallas import tpu as pltpu
from jax.experimental.pallas import tpu_sc as plsc

D = 128
P = 256
N = 4096
NT = N + P
E = 65536

BLK = 256
NBLK = NT // BLK

NC = 2
NS = 16
L = 16
NW = NC * NS
CH = E // NW
ROWS = CH // 128


def _dense_body(f_ref, ft_ref, o_ref):
    i = pl.program_id(0)
    f = f_ref[...]
    right = lax.dot_general(f, ft_ref[:, N:],
                            (((1,), (0,)), ((), ())),
                            preferred_element_type=jnp.float32)
    o_ref[:, N:] = jax.nn.sigmoid(right)

    @pl.when(i < NBLK - 1)
    def _zero():
        o_ref[:, :N] = jnp.zeros((BLK, N), jnp.float32)

    @pl.when(i == NBLK - 1)
    def _bottom():
        left = lax.dot_general(f, ft_ref[:, :N],
                               (((1,), (0,)), ((), ())),
                               preferred_element_type=jnp.float32)
        o_ref[:, :N] = jax.nn.sigmoid(left)


_dense = pl.pallas_call(
    _dense_body,
    grid=(NBLK,),
    in_specs=[
        pl.BlockSpec((BLK, D), lambda i: (i, 0)),
        pl.BlockSpec((D, NT), lambda i: (0, 0)),
    ],
    out_specs=pl.BlockSpec((BLK, NT), lambda i: (i, 0)),
    out_shape=jax.ShapeDtypeStruct((NT, NT), jnp.float32),
    compiler_params=pltpu.CompilerParams(
        dimension_semantics=("arbitrary",)),
)


@functools.partial(
    pl.kernel,
    mesh=plsc.VectorSubcoreMesh(core_axis_name="c", subcore_axis_name="s"),
    scratch_types=[
        pltpu.VMEM((CH,), jnp.int32),
        pltpu.VMEM((CH,), jnp.int32),
        pltpu.VMEM((CH,), jnp.int32),
        pltpu.VMEM((CH,), jnp.float32),
        pltpu.SemaphoreType.DMA,
    ],
)
def _scatter(src_hbm, dst_hbm, out_hbm, src_v, dst_v, idx_v, ones_v, sem):
    wid = lax.axis_index("s") * NC + lax.axis_index("c")
    base = wid * CH
    pltpu.sync_copy(src_hbm.at[pl.ds(base, CH)], src_v)
    pltpu.sync_copy(dst_hbm.at[pl.ds(base, CH)], dst_v)
    for k in range(CH // L):
        s = src_v[pl.ds(k * L, L)]
        t = dst_v[pl.ds(k * L, L)]
        flat = (
            ((s >> 3) * (34 * 1024) + (t >> 7) * 1024)
            + ((s & 7) << 7)
            + (t & 127)
        )
        idx_v[pl.ds(k * L, L)] = flat
        ones_v[pl.ds(k * L, L)] = jnp.ones((L,), jnp.float32)
    pltpu.async_copy(ones_v, out_hbm.at[idx_v], sem).wait()


def kernel(x, edge_index, super_nodes):
    f = jnp.concatenate([x, super_nodes], axis=0)
    ft = f.T
    dense = _dense(f, ft)
    src = edge_index[0].astype(jnp.int32)
    dst = edge_index[1].astype(jnp.int32)
    d4 = dense.reshape(NT // 8, 8, NT // 128, 128).transpose(0, 2, 1, 3)
    out_ref = jax.new_ref(d4.reshape(NT * NT))
    _scatter(src, dst, out_ref)
    out4 = out_ref[...].reshape(NT // 8, NT // 128, 8, 128)
    return out4.transpose(0, 2, 1, 3).reshape(NT, NT)


# speedup vs baseline: 1.0642x; 1.0642x over previous
import functools

import jax
import jax.numpy as jnp
from jax import lax
from jax.experimental import pallas as pl
from jax.experimental.pallas import tpu as pltpu
from jax.experimental.pallas import tpu_sc as plsc

D = 128
P = 256
N = 4096
NT = N + P
E = 65536

BLK = 256
NBLK = NT // BLK

NC = 2
NS = 16
L = 16
NW = NC * NS
CH = E // NW
ROWS = CH // 128


def _sigmoid(v):
    return 0.5 * jnp.tanh(0.5 * v) + 0.5


def _dense_body(x_blk_ref, x_full_ref, sn_ref, o_ref):
    i = pl.program_id(0)
    sn = sn_ref[...]
    f = jnp.where(i == NBLK - 1, sn, x_blk_ref[...])
    right = lax.dot_general(f, sn,
                            (((1,), (1,)), ((), ())),
                            preferred_element_type=jnp.float32)
    o_ref[:, N:] = _sigmoid(right)

    @pl.when(i < NBLK - 1)
    def _zero():
        o_ref[:, :N] = jnp.zeros((BLK, N), jnp.float32)

    @pl.when(i == NBLK - 1)
    def _bottom():
        left = lax.dot_general(sn, x_full_ref[...],
                               (((1,), (1,)), ((), ())),
                               preferred_element_type=jnp.float32)
        o_ref[:, :N] = _sigmoid(left)


_dense = pl.pallas_call(
    _dense_body,
    grid=(NBLK,),
    in_specs=[
        pl.BlockSpec((BLK, D), lambda i: (jnp.minimum(i, NBLK - 2), 0)),
        pl.BlockSpec((N, D), lambda i: (0, 0)),
        pl.BlockSpec((P, D), lambda i: (0, 0)),
    ],
    out_specs=pl.BlockSpec((BLK, NT), lambda i: (i, 0)),
    out_shape=jax.ShapeDtypeStruct((NT, NT), jnp.float32),
    compiler_params=pltpu.CompilerParams(
        dimension_semantics=("arbitrary",)),
)


@functools.partial(
    pl.kernel,
    mesh=plsc.VectorSubcoreMesh(core_axis_name="c", subcore_axis_name="s"),
    scratch_types=[
        pltpu.VMEM((CH,), jnp.int32),
        pltpu.VMEM((CH,), jnp.int32),
        pltpu.VMEM((CH,), jnp.int32),
        pltpu.VMEM((CH,), jnp.float32),
        pltpu.SemaphoreType.DMA,
    ],
)
def _scatter(src_hbm, dst_hbm, out_hbm, src_v, dst_v, idx_v, ones_v, sem):
    wid = lax.axis_index("s") * NC + lax.axis_index("c")
    base = wid * CH
    pltpu.sync_copy(src_hbm.at[pl.ds(base, CH)], src_v)
    pltpu.sync_copy(dst_hbm.at[pl.ds(base, CH)], dst_v)
    for k in range(CH // L):
        s = src_v[pl.ds(k * L, L)]
        t = dst_v[pl.ds(k * L, L)]
        flat = (
            ((s >> 3) * (34 * 1024) + (t >> 7) * 1024)
            + ((s & 7) << 7)
            + (t & 127)
        )
        idx_v[pl.ds(k * L, L)] = flat
        ones_v[pl.ds(k * L, L)] = jnp.ones((L,), jnp.float32)
    pltpu.async_copy(ones_v, out_hbm.at[idx_v], sem).wait()


def kernel(x, edge_index, super_nodes):
    dense = _dense(x, x, super_nodes)
    src = edge_index[0].astype(jnp.int32)
    dst = edge_index[1].astype(jnp.int32)
    d4 = dense.reshape(NT // 8, 8, NT // 128, 128).transpose(0, 2, 1, 3)
    out_ref = jax.new_ref(d4.reshape(NT * NT))
    _scatter(src, dst, out_ref)
    out4 = out_ref[...].reshape(NT // 8, NT // 128, 8, 128)
    return out4.transpose(0, 2, 1, 3).reshape(NT, NT)


# speedup vs baseline: 1.0901x; 1.0243x over previous
import functools

import jax
import jax.numpy as jnp
from jax import lax
from jax.experimental import pallas as pl
from jax.experimental.pallas import tpu as pltpu
from jax.experimental.pallas import tpu_sc as plsc

D = 128
P = 256
N = 4096
NT = N + P
E = 65536

BLK = 256
NBLK = NT // BLK

NC = 2
NS = 16
L = 16
NW = NC * NS
CH = E // NW
ROWS = CH // 128


def _sigmoid(v):
    return 0.5 * jnp.tanh(0.5 * v) + 0.5


def _dense_body(x_blk_ref, x_full_ref, sn_ref, o_ref):
    i = pl.program_id(0)
    sn = sn_ref[...]
    f = jnp.where(i == NBLK - 1, sn, x_blk_ref[...])
    right = lax.dot_general(f, sn,
                            (((1,), (1,)), ((), ())),
                            preferred_element_type=jnp.float32)
    o_ref[:, N:] = _sigmoid(right)

    @pl.when(i < NBLK - 1)
    def _zero():
        o_ref[:, :N] = jnp.zeros((BLK, N), jnp.float32)

    @pl.when(i == NBLK - 1)
    def _bottom():
        left = lax.dot_general(sn, x_full_ref[...],
                               (((1,), (1,)), ((), ())),
                               preferred_element_type=jnp.float32)
        o_ref[:, :N] = _sigmoid(left)


_dense = pl.pallas_call(
    _dense_body,
    grid=(NBLK,),
    in_specs=[
        pl.BlockSpec((BLK, D), lambda i: (jnp.minimum(i, NBLK - 2), 0)),
        pl.BlockSpec((N, D), lambda i: (0, 0)),
        pl.BlockSpec((P, D), lambda i: (0, 0)),
    ],
    out_specs=pl.BlockSpec((BLK, NT), lambda i: (i, 0)),
    out_shape=jax.ShapeDtypeStruct((NT, NT), jnp.float32),
    compiler_params=pltpu.CompilerParams(
        dimension_semantics=("arbitrary",)),
)


@functools.partial(
    pl.kernel,
    mesh=plsc.VectorSubcoreMesh(core_axis_name="c", subcore_axis_name="s"),
    scratch_types=[
        pltpu.VMEM((CH,), jnp.int32),
        pltpu.VMEM((CH,), jnp.int32),
        pltpu.VMEM((CH,), jnp.int32),
        pltpu.VMEM((CH,), jnp.float32),
        pltpu.SemaphoreType.DMA,
    ],
)
def _scatter(edge_hbm, out_hbm, src_v, dst_v, idx_v, ones_v, sem):
    wid = lax.axis_index("s") * NC + lax.axis_index("c")
    base = wid * CH
    pltpu.sync_copy(edge_hbm.at[0, pl.ds(base, CH)], src_v)
    pltpu.sync_copy(edge_hbm.at[1, pl.ds(base, CH)], dst_v)
    for k in range(CH // L):
        s = src_v[pl.ds(k * L, L)]
        t = dst_v[pl.ds(k * L, L)]
        flat = (
            ((s >> 3) * (34 * 1024) + (t >> 7) * 1024)
            + ((s & 7) << 7)
            + (t & 127)
        )
        idx_v[pl.ds(k * L, L)] = flat
        ones_v[pl.ds(k * L, L)] = jnp.ones((L,), jnp.float32)
    pltpu.async_copy(ones_v, out_hbm.at[idx_v], sem).wait()


def kernel(x, edge_index, super_nodes):
    dense = _dense(x, x, super_nodes)
    edges = edge_index.astype(jnp.int32)
    d4 = dense.reshape(NT // 8, 8, NT // 128, 128).transpose(0, 2, 1, 3)
    out_ref = jax.new_ref(d4.reshape(NT * NT))
    _scatter(edges, out_ref)
    out4 = out_ref[...].reshape(NT // 8, NT // 128, 8, 128)
    return out4.transpose(0, 2, 1, 3).reshape(NT, NT)


# speedup vs baseline: 1.0928x; 1.0025x over previous
import functools

import jax
import jax.numpy as jnp
from jax import lax
from jax.experimental import pallas as pl
from jax.experimental.pallas import tpu as pltpu
from jax.experimental.pallas import tpu_sc as plsc

D = 128
P = 256
N = 4096
NT = N + P
E = 65536

BLK = 256
NBLK = NT // BLK

NC = 2
NS = 16
L = 16
NW = NC * NS
CH = E // NW
ROWS = CH // 128


def _sigmoid(v):
    return 0.5 * jnp.tanh(0.5 * v) + 0.5


def _dense_body(x_blk_ref, x_full_ref, sn_ref, o_ref):
    i = pl.program_id(0)
    sn = sn_ref[...]
    f = jnp.where(i == NBLK - 1, sn, x_blk_ref[...])
    right = lax.dot_general(f, sn,
                            (((1,), (1,)), ((), ())),
                            preferred_element_type=jnp.float32)
    o_ref[:, N:] = _sigmoid(right)

    @pl.when(i < NBLK - 1)
    def _zero():
        o_ref[:, :N] = jnp.zeros((BLK, N), jnp.float32)

    @pl.when(i == NBLK - 1)
    def _bottom():
        left = lax.dot_general(sn, x_full_ref[...],
                               (((1,), (1,)), ((), ())),
                               preferred_element_type=jnp.float32)
        o_ref[:, :N] = _sigmoid(left)


_dense = pl.pallas_call(
    _dense_body,
    grid=(NBLK,),
    in_specs=[
        pl.BlockSpec((BLK, D), lambda i: (jnp.minimum(i, NBLK - 2), 0)),
        pl.BlockSpec((N, D), lambda i: (0, 0)),
        pl.BlockSpec((P, D), lambda i: (0, 0)),
    ],
    out_specs=pl.BlockSpec((BLK, NT), lambda i: (i, 0)),
    out_shape=jax.ShapeDtypeStruct((NT, NT), jnp.float32),
    compiler_params=pltpu.CompilerParams(
        dimension_semantics=("arbitrary",)),
)


@functools.partial(
    pl.kernel,
    mesh=plsc.VectorSubcoreMesh(core_axis_name="c", subcore_axis_name="s"),
    scratch_types=[
        pltpu.VMEM((CH,), jnp.int32),
        pltpu.VMEM((CH,), jnp.int32),
        pltpu.VMEM((ROWS, 128), jnp.int32),
        pltpu.VMEM((ROWS, 128), jnp.float32),
        pltpu.SemaphoreType.DMA,
    ],
)
def _scatter(edge_hbm, out_hbm, src_v, dst_v, idx_v, ones_v, sem):
    wid = lax.axis_index("s") * NC + lax.axis_index("c")
    base = wid * CH
    pltpu.sync_copy(edge_hbm.at[0, pl.ds(base, CH)], src_v)
    pltpu.sync_copy(edge_hbm.at[1, pl.ds(base, CH)], dst_v)
    copies = []
    for j in range(ROWS):
        for c in range(128 // L):
            k = j * (128 // L) + c
            s = src_v[pl.ds(k * L, L)]
            t = dst_v[pl.ds(k * L, L)]
            flat = (
                ((s >> 3) * (34 * 1024) + (t >> 7) * 1024)
                + ((s & 7) << 7)
                + (t & 127)
            )
            idx_v[j, pl.ds(c * L, L)] = flat
            ones_v[j, pl.ds(c * L, L)] = jnp.ones((L,), jnp.float32)
        copies.append(
            pltpu.async_copy(ones_v.at[j], out_hbm.at[idx_v.at[j]], sem)
        )
    for cp in copies:
        cp.wait()


def kernel(x, edge_index, super_nodes):
    dense = _dense(x, x, super_nodes)
    edges = edge_index.astype(jnp.int32)
    d4 = dense.reshape(NT // 8, 8, NT // 128, 128).transpose(0, 2, 1, 3)
    out_ref = jax.new_ref(d4.reshape(NT * NT))
    _scatter(edges, out_ref)
    out4 = out_ref[...].reshape(NT // 8, NT // 128, 8, 128)
    return out4.transpose(0, 2, 1, 3).reshape(NT, NT)


# speedup vs baseline: 1.1051x; 1.0112x over previous
import functools

import jax
import jax.numpy as jnp
from jax import lax
from jax.experimental import pallas as pl
from jax.experimental.pallas import tpu as pltpu
from jax.experimental.pallas import tpu_sc as plsc

D = 128
P = 256
N = 4096
NT = N + P
E = 65536

BLK = 256
NBLK = NT // BLK

NC = 2
NS = 16
L = 16
NW = NC * NS
CH = E // NW
ROWS = CH // 128


def _sigmoid(v):
    return 0.5 * jnp.tanh(0.5 * v) + 0.5


def _dense_body(x_blk_ref, sn_ref, o_ref, bot_ref):
    i = pl.program_id(0)
    sn = sn_ref[...]
    f = jnp.where(i == NBLK - 1, sn, x_blk_ref[...])
    right = _sigmoid(lax.dot_general(f, sn,
                                     (((1,), (1,)), ((), ())),
                                     preferred_element_type=jnp.float32))
    o_ref[:, N:] = right
    bot_ref[:, pl.ds(i * BLK, BLK)] = right.T

    @pl.when(i < NBLK - 1)
    def _zero():
        o_ref[:, :N] = jnp.zeros((BLK, N), jnp.float32)

    @pl.when(i == NBLK - 1)
    def _bottom():
        o_ref[:, :N] = bot_ref[:, :N]


_dense = pl.pallas_call(
    _dense_body,
    grid=(NBLK,),
    in_specs=[
        pl.BlockSpec((BLK, D), lambda i: (jnp.minimum(i, NBLK - 2), 0)),
        pl.BlockSpec((P, D), lambda i: (0, 0)),
    ],
    out_specs=pl.BlockSpec((BLK, NT), lambda i: (i, 0)),
    out_shape=jax.ShapeDtypeStruct((NT, NT), jnp.float32),
    scratch_shapes=[pltpu.VMEM((P, NT), jnp.float32)],
    compiler_params=pltpu.CompilerParams(
        dimension_semantics=("arbitrary",)),
)


@functools.partial(
    pl.kernel,
    mesh=plsc.VectorSubcoreMesh(core_axis_name="c", subcore_axis_name="s"),
    scratch_types=[
        pltpu.VMEM((CH,), jnp.int32),
        pltpu.VMEM((CH,), jnp.int32),
        pltpu.VMEM((ROWS, 128), jnp.int32),
        pltpu.VMEM((ROWS, 128), jnp.float32),
        pltpu.SemaphoreType.DMA,
    ],
)
def _scatter(edge_hbm, out_hbm, src_v, dst_v, idx_v, ones_v, sem):
    wid = lax.axis_index("s") * NC + lax.axis_index("c")
    base = wid * CH
    pltpu.sync_copy(edge_hbm.at[0, pl.ds(base, CH)], src_v)
    pltpu.sync_copy(edge_hbm.at[1, pl.ds(base, CH)], dst_v)
    copies = []
    for j in range(ROWS):
        for c in range(128 // L):
            k = j * (128 // L) + c
            s = src_v[pl.ds(k * L, L)]
            t = dst_v[pl.ds(k * L, L)]
            flat = (
                ((s >> 3) * (34 * 1024) + (t >> 7) * 1024)
                + ((s & 7) << 7)
                + (t & 127)
            )
            idx_v[j, pl.ds(c * L, L)] = flat
            ones_v[j, pl.ds(c * L, L)] = jnp.ones((L,), jnp.float32)
        copies.append(
            pltpu.async_copy(ones_v.at[j], out_hbm.at[idx_v.at[j]], sem)
        )
    for cp in copies:
        cp.wait()


def kernel(x, edge_index, super_nodes):
    dense = _dense(x, super_nodes)
    edges = edge_index.astype(jnp.int32)
    d4 = dense.reshape(NT // 8, 8, NT // 128, 128).transpose(0, 2, 1, 3)
    out_ref = jax.new_ref(d4.reshape(NT * NT))
    _scatter(edges, out_ref)
    out4 = out_ref[...].reshape(NT // 8, NT // 128, 8, 128)
    return out4.transpose(0, 2, 1, 3).reshape(NT, NT)


# speedup vs baseline: 1.1058x; 1.0007x over previous
import functools

import jax
import jax.numpy as jnp
from jax import lax
from jax.experimental import pallas as pl
from jax.experimental.pallas import tpu as pltpu
from jax.experimental.pallas import tpu_sc as plsc

D = 128
P = 256
N = 4096
NT = N + P
E = 65536

BLK = 256
NBLK = NT // BLK

NC = 2
NS = 16
L = 16
NW = NC * NS
CH = E // NW
ROWS = CH // 128


def _sigmoid(v):
    return 0.5 * jnp.tanh(0.5 * v) + 0.5


def _dense_body(x_blk_ref, sn_ref, o_ref, bot_ref):
    i = pl.program_id(0)
    sn = sn_ref[...]
    f = jnp.where(i == NBLK - 1, sn, x_blk_ref[...])
    right = _sigmoid(lax.dot_general(f, sn,
                                     (((1,), (1,)), ((), ())),
                                     preferred_element_type=jnp.float32))
    o_ref[:, N:] = right
    bot_ref[:, pl.ds(i * BLK, BLK)] = right.T

    @pl.when(i < NBLK - 1)
    def _zero():
        o_ref[:, :N] = jnp.zeros((BLK, N), jnp.float32)

    @pl.when(i == NBLK - 1)
    def _bottom():
        o_ref[:, :N] = bot_ref[:, :N]


_dense = pl.pallas_call(
    _dense_body,
    grid=(NBLK,),
    in_specs=[
        pl.BlockSpec((BLK, D), lambda i: (jnp.minimum(i, NBLK - 2), 0)),
        pl.BlockSpec((P, D), lambda i: (0, 0)),
    ],
    out_specs=pl.BlockSpec((BLK, NT), lambda i: (i, 0)),
    out_shape=jax.ShapeDtypeStruct((NT, NT), jnp.float32),
    scratch_shapes=[pltpu.VMEM((P, NT), jnp.float32)],
    compiler_params=pltpu.CompilerParams(
        dimension_semantics=("arbitrary",)),
)


@functools.partial(
    pl.kernel,
    mesh=plsc.VectorSubcoreMesh(core_axis_name="c", subcore_axis_name="s"),
    scratch_types=[
        pltpu.VMEM((2, CH), jnp.int32),
        pltpu.VMEM((ROWS, 128), jnp.int32),
        pltpu.VMEM((ROWS, 128), jnp.float32),
        pltpu.SemaphoreType.DMA,
    ],
)
def _scatter(edge_hbm, out_hbm, edge_v, idx_v, ones_v, sem):
    wid = lax.axis_index("s") * NC + lax.axis_index("c")
    base = wid * CH
    pltpu.sync_copy(edge_hbm.at[:, pl.ds(base, CH)], edge_v)
    copies = []
    for j in range(ROWS):
        for c in range(128 // L):
            k = j * (128 // L) + c
            s = edge_v[0, pl.ds(k * L, L)]
            t = edge_v[1, pl.ds(k * L, L)]
            flat = (
                ((s >> 3) * (34 * 1024) + (t >> 7) * 1024)
                + ((s & 7) << 7)
                + (t & 127)
            )
            idx_v[j, pl.ds(c * L, L)] = flat
            ones_v[j, pl.ds(c * L, L)] = jnp.ones((L,), jnp.float32)
        copies.append(
            pltpu.async_copy(ones_v.at[j], out_hbm.at[idx_v.at[j]], sem)
        )
    for cp in copies:
        cp.wait()


def kernel(x, edge_index, super_nodes):
    dense = _dense(x, super_nodes)
    edges = edge_index.astype(jnp.int32)
    d4 = dense.reshape(NT // 8, 8, NT // 128, 128).transpose(0, 2, 1, 3)
    out_ref = jax.new_ref(d4.reshape(NT * NT))
    _scatter(edges, out_ref)
    out4 = out_ref[...].reshape(NT // 8, NT // 128, 8, 128)
    return out4.transpose(0, 2, 1, 3).reshape(NT, NT)
